# baseline (speedup 1.0000x reference)
import functools

import jax
import jax.numpy as jnp
from jax import lax
from jax.experimental import pallas as pl
from jax.experimental.pallas import tpu as pltpu
from jax.experimental.pallas import tpu_sc as plsc


@functools.cache
def _build(H, NBATCH, D):
    info = plsc.get_sparse_core_info()
    NC, NS = info.num_cores, info.num_subcores
    NW = NC * NS
    BK = 128
    TB = NBATCH // BK
    NBLK = H * TB
    nb_w = NBLK // NW
    n_idx_w = nb_w * BK
    TD, DD = D // 8, 8
    assert nb_w * NW == NBLK and nb_w % 4 == 0

    mesh = plsc.VectorSubcoreMesh(core_axis_name="c", subcore_axis_name="s")

    @functools.partial(
        pl.kernel,
        mesh=mesh,
        out_type=jax.ShapeDtypeStruct((H, TD, TB, DD, BK), jnp.float32),
        scratch_types=[
            pltpu.VMEM((n_idx_w,), jnp.int32),
            pltpu.VMEM((BK, D), jnp.float32),
            pltpu.VMEM((BK, D), jnp.float32),
            pltpu.VMEM((BK, D), jnp.float32),
            pltpu.VMEM((BK, D), jnp.float32),
            pltpu.VMEM((TD, DD, BK), jnp.float32),
            pltpu.VMEM((TD, DD, BK), jnp.float32),
            pltpu.SemaphoreType.DMA,
            pltpu.SemaphoreType.DMA,
            pltpu.SemaphoreType.DMA,
            pltpu.SemaphoreType.DMA,
            pltpu.SemaphoreType.DMA,
            pltpu.SemaphoreType.DMA,
        ],
        compiler_params=pltpu.CompilerParams(
            use_tc_tiling_on_sc=False, needs_layout_passes=False),
    )
    def gather_kernel(idx_hbm, table_hbm, out_hbm, idx_v,
                      gb0, gb1, gb2, gb3, tb0, tb1,
                      g0, g1, g2, g3, s0, s1):
        gbufs = (gb0, gb1, gb2, gb3)
        gsems = (g0, g1, g2, g3)
        tbufs = (tb0, tb1)
        ssems = (s0, s1)

        wid = lax.axis_index("s") * NC + lax.axis_index("c")
        base = wid * n_idx_w
        pltpu.sync_copy(idx_hbm.at[pl.ds(base, n_idx_w)], idx_v)

        def gather(j, gb, gs):
            return pltpu.make_async_copy(
                table_hbm.at[idx_v.at[pl.ds(j * BK, BK)]], gb, gs)

        def dst(j):
            k = wid * nb_w + j
            return out_hbm.at[k // TB, pl.ds(0, TD), k % TB]

        iota = lax.iota(jnp.int32, 16)
        rows = [iota + 16 * jj for jj in range(8)]

        def transpose(gb, tb):
            @pl.loop(0, D, step=8)
            def _(d0):
                for dd in range(8):
                    d = d0 + dd
                    cols = jnp.zeros((16,), jnp.int32) + d
                    td = d // DD
                    dj = d % DD
                    for jj in range(8):
                        v = plsc.load_gather(gb, [rows[jj], cols])
                        tb[td, dj, pl.ds(16 * jj, 16)] = v

        for b in range(4):
            gather(b, gbufs[b], gsems[b]).start()

        @pl.loop(0, nb_w, step=4)
        def _(j0):
            for b in range(4):
                j = j0 + b
                gather(j, gbufs[b], gsems[b]).wait()

                @pl.when(j >= 2)
                def _():
                    pltpu.make_async_copy(
                        tbufs[b % 2], dst(j - 2), ssems[b % 2]).wait()

                transpose(gbufs[b], tbufs[b % 2])

                @pl.when(j + 4 < nb_w)
                def _():
                    gather(j + 4, gbufs[b], gsems[b]).start()

                pltpu.make_async_copy(
                    tbufs[b % 2], dst(j), ssems[b % 2]).start()

        pltpu.make_async_copy(tbufs[0], dst(nb_w - 2), ssems[0]).wait()
        pltpu.make_async_copy(tbufs[1], dst(nb_w - 1), ssems[1]).wait()

    return gather_kernel


def kernel(x, word_vectors):
    NBATCH, H = x.shape
    D = word_vectors.shape[1]
    idx = x.T.reshape(NBATCH * H)
    wv_wide = lax.optimization_barrier(
        word_vectors.reshape(word_vectors.shape[0] // 2, 2 * D))
    wv_lin = wv_wide.reshape(word_vectors.shape[0], D)
    out5 = _build(H, NBATCH, D)(idx, wv_lin)
    return out5.transpose(2, 4, 0, 1, 3).reshape(NBATCH, H, D)


# speedup vs baseline: 1.8591x; 1.8591x over previous
import functools

import jax
import jax.numpy as jnp
from jax import lax
from jax.experimental import pallas as pl
from jax.experimental.pallas import tpu as pltpu
from jax.experimental.pallas import tpu_sc as plsc


@functools.cache
def _build(H, NBATCH, D):
    info = plsc.get_sparse_core_info()
    NC, NS = info.num_cores, info.num_subcores
    NW = NC * NS
    BK = 128
    TB = NBATCH // BK
    NBLK = H * TB
    nb_w = NBLK // NW
    n_idx_w = nb_w * BK
    TD, DD = D // 8, 8
    assert nb_w * NW == NBLK and nb_w % 4 == 0

    mesh = plsc.VectorSubcoreMesh(core_axis_name="c", subcore_axis_name="s")

    @functools.partial(
        pl.kernel,
        mesh=mesh,
        out_type=jax.ShapeDtypeStruct((H, TD, TB, DD, BK), jnp.float32),
        scratch_types=[
            pltpu.VMEM((n_idx_w,), jnp.int32),
            pltpu.VMEM((BK, D), jnp.float32),
            pltpu.VMEM((BK, D), jnp.float32),
            pltpu.VMEM((BK, D), jnp.float32),
            pltpu.VMEM((BK, D), jnp.float32),
            pltpu.VMEM((TD, DD, BK + 1), jnp.float32),
            pltpu.VMEM((TD, DD, BK + 1), jnp.float32),
            pltpu.SemaphoreType.DMA,
            pltpu.SemaphoreType.DMA,
            pltpu.SemaphoreType.DMA,
            pltpu.SemaphoreType.DMA,
            pltpu.SemaphoreType.DMA,
            pltpu.SemaphoreType.DMA,
        ],
        compiler_params=pltpu.CompilerParams(
            use_tc_tiling_on_sc=False, needs_layout_passes=False),
    )
    def gather_kernel(idx_hbm, table_hbm, out_hbm, idx_v,
                      gb0, gb1, gb2, gb3, tb0, tb1,
                      g0, g1, g2, g3, s0, s1):
        gbufs = (gb0, gb1, gb2, gb3)
        gsems = (g0, g1, g2, g3)
        tbufs = (tb0, tb1)
        ssems = (s0, s1)

        wid = lax.axis_index("s") * NC + lax.axis_index("c")
        base = wid * n_idx_w
        pltpu.sync_copy(idx_hbm.at[pl.ds(base, n_idx_w)], idx_v)

        def gather(j, gb, gs):
            return pltpu.make_async_copy(
                table_hbm.at[idx_v.at[pl.ds(j * BK, BK)]], gb, gs)

        def dst(j):
            k = wid * nb_w + j
            return out_hbm.at[k // TB, pl.ds(0, TD), k % TB]

        iota = lax.iota(jnp.int32, 16)
        tdv = [(16 * q + iota) // DD for q in range(D // 16)]
        ddv = [lax.rem(16 * q + iota, DD) for q in range(D // 16)]

        def transpose(gb, tb):
            @pl.loop(0, BK, step=4)
            def _(b0):
                for bo in range(4):
                    b = b0 + bo
                    bs = jnp.zeros((16,), jnp.int32) + b
                    for q in range(D // 16):
                        v = gb[b, pl.ds(16 * q, 16)]
                        plsc.store_scatter(tb, [tdv[q], ddv[q], bs], v)

        for b in range(4):
            gather(b, gbufs[b], gsems[b]).start()

        @pl.loop(0, nb_w, step=4)
        def _(j0):
            for b in range(4):
                j = j0 + b
                gather(j, gbufs[b], gsems[b]).wait()

                @pl.when(j >= 2)
                def _():
                    pltpu.make_async_copy(
                        tbufs[b % 2].at[pl.ds(0, TD), pl.ds(0, DD),
                                        pl.ds(0, BK)],
                        dst(j - 2), ssems[b % 2]).wait()

                transpose(gbufs[b], tbufs[b % 2])

                @pl.when(j + 4 < nb_w)
                def _():
                    gather(j + 4, gbufs[b], gsems[b]).start()

                pltpu.make_async_copy(
                    tbufs[b % 2].at[pl.ds(0, TD), pl.ds(0, DD),
                                    pl.ds(0, BK)],
                    dst(j), ssems[b % 2]).start()

        pltpu.make_async_copy(
            tbufs[0].at[pl.ds(0, TD), pl.ds(0, DD), pl.ds(0, BK)],
            dst(nb_w - 2), ssems[0]).wait()
        pltpu.make_async_copy(
            tbufs[1].at[pl.ds(0, TD), pl.ds(0, DD), pl.ds(0, BK)],
            dst(nb_w - 1), ssems[1]).wait()

    return gather_kernel


def kernel(x, word_vectors):
    NBATCH, H = x.shape
    D = word_vectors.shape[1]
    idx = x.T.reshape(NBATCH * H)
    wv_wide = lax.optimization_barrier(
        word_vectors.reshape(word_vectors.shape[0] // 2, 2 * D))
    wv_lin = wv_wide.reshape(word_vectors.shape[0], D)
    out5 = _build(H, NBATCH, D)(idx, wv_lin)
    return out5.transpose(2, 4, 0, 1, 3).reshape(NBATCH, H, D)
